# initial kernel scaffold (unmeasured)
import jax
import jax.numpy as jnp
from jax import lax
from jax.experimental import pallas as pl
from jax.experimental.pallas import tpu as pltpu

P = 16
E_LOCAL = 2
N_EXPERTS = 32


def kernel(x, router_W, route_idx, expert_W):
    n, d = x.shape
    h = expert_W.shape[-1]
    r = n // P

    def body(x_ref, rw_ref, idx_ref, ew_ref, out_ref,
             acc_ref, recv_ref, send_sems, recv_sems):
        my = lax.axis_index("i")

        xv = x_ref[:, :]
        scores = jnp.dot(xv, rw_ref[:, :], preferred_element_type=jnp.float32)
        m = jnp.max(scores, axis=-1, keepdims=True)
        p = jnp.exp(scores - m)
        p = p / jnp.sum(p, axis=-1, keepdims=True)
        idx0 = idx_ref[:, 0:1]
        idx1 = idx_ref[:, 1:2]
        e_iota = lax.broadcasted_iota(jnp.int32, (n, N_EXPERTS), 1)
        g0 = jnp.sum(jnp.where(idx0 == e_iota, p, 0.0), axis=1, keepdims=True)
        g1 = jnp.sum(jnp.where(idx1 == e_iota, p, 0.0), axis=1, keepdims=True)
        gsum = g0 + g1
        g0n = g0 / gsum
        g1n = g1 / gsum

        partial = jnp.zeros((n, h), jnp.float32)
        for le in range(E_LOCAL):
            ge = my * E_LOCAL + le
            w = jnp.where(idx0 == ge, g0n, 0.0) + jnp.where(idx1 == ge, g1n, 0.0)
            y = jnp.dot(xv, ew_ref[le, :, :], preferred_element_type=jnp.float32)
            partial = partial + w * y

        acc_ref[:, :, :] = partial.reshape(P, r, h)
        recv_ref[0, :, :] = jnp.zeros((r, h), jnp.float32)

        bsem = pltpu.get_barrier_semaphore()
        for j in range(1, P):
            t = lax.rem(my + j, P)
            pl.semaphore_signal(bsem, inc=1, device_id=(t,),
                                device_id_type=pl.DeviceIdType.MESH)
        pl.semaphore_wait(bsem, P - 1)

        rdmas = []
        for j in range(1, P):
            t = lax.rem(my + j, P)
            rd = pltpu.make_async_remote_copy(
                src_ref=acc_ref.at[pl.ds(t, 1)],
                dst_ref=recv_ref.at[pl.ds(j, 1)],
                send_sem=send_sems.at[j],
                recv_sem=recv_sems.at[j],
                device_id=(t,),
                device_id_type=pl.DeviceIdType.MESH,
            )
            rd.start()
            rdmas.append(rd)
        for rd in rdmas:
            rd.wait_recv()

        own = lax.dynamic_slice(partial, (my * r, 0), (r, h))
        out_ref[:, :] = own + jnp.sum(recv_ref[:, :, :], axis=0)
        for rd in rdmas:
            rd.wait_send()

    return pl.pallas_call(
        body,
        out_shape=jax.ShapeDtypeStruct((r, h), jnp.float32),
        in_specs=[pl.BlockSpec(memory_space=pltpu.VMEM)] * 4,
        out_specs=pl.BlockSpec(memory_space=pltpu.VMEM),
        scratch_shapes=[
            pltpu.VMEM((P, r, h), jnp.float32),
            pltpu.VMEM((P, r, h), jnp.float32),
            pltpu.SemaphoreType.DMA((P,)),
            pltpu.SemaphoreType.DMA((P,)),
        ],
        compiler_params=pltpu.CompilerParams(collective_id=0),
    )(x, router_W, route_idx, expert_W)


# baseline (device time: 12986 ns/iter reference)
import jax
import jax.numpy as jnp
from jax import lax
from jax.experimental import pallas as pl
from jax.experimental.pallas import tpu as pltpu

P = 16
E_LOCAL = 2
N_EXPERTS = 32


def kernel(x, router_W, route_idx, expert_W):
    n, d = x.shape
    h = expert_W.shape[-1]
    r = n // P

    def body(x_ref, rw_ref, idx_ref, ew_ref, out_ref,
             acc_ref, recv_ref, send_sems, recv_sems):
        my = lax.axis_index("i")

        xv = x_ref[:, :]
        scores = jnp.dot(xv, rw_ref[:, :], preferred_element_type=jnp.float32)
        m = jnp.max(scores, axis=-1, keepdims=True)
        p = jnp.exp(scores - m)
        p = p / jnp.sum(p, axis=-1, keepdims=True)
        idx0 = idx_ref[:, 0:1]
        idx1 = idx_ref[:, 1:2]
        e_iota = lax.broadcasted_iota(jnp.int32, (n, N_EXPERTS), 1)
        g0 = jnp.sum(jnp.where(idx0 == e_iota, p, 0.0), axis=1, keepdims=True)
        g1 = jnp.sum(jnp.where(idx1 == e_iota, p, 0.0), axis=1, keepdims=True)
        gsum = g0 + g1
        g0n = g0 / gsum
        g1n = g1 / gsum

        partial = jnp.zeros((n, h), jnp.float32)
        for le in range(E_LOCAL):
            ge = my * E_LOCAL + le
            w = jnp.where(idx0 == ge, g0n, 0.0) + jnp.where(idx1 == ge, g1n, 0.0)
            y = jnp.dot(xv, ew_ref[le, :, :], preferred_element_type=jnp.float32)
            partial = partial + w * y

        acc_ref[:, :, :] = partial.reshape(P, r, h)
        recv_ref[0, :, :] = jnp.zeros((r, h), jnp.float32)

        bsem = pltpu.get_barrier_semaphore()
        for j in range(1, P):
            t = lax.rem(my + j, P)
            pl.semaphore_signal(bsem, inc=1, device_id=(t,),
                                device_id_type=pl.DeviceIdType.MESH)
        pl.semaphore_wait(bsem, P - 1)

        rdmas = []
        for j in range(1, P):
            t = lax.rem(my + j, P)
            rd = pltpu.make_async_remote_copy(
                src_ref=acc_ref.at[pl.ds(t, 1)],
                dst_ref=recv_ref.at[pl.ds(j, 1)],
                send_sem=send_sems.at[j],
                recv_sem=recv_sems.at[j],
                device_id=(t,),
                device_id_type=pl.DeviceIdType.MESH,
            )
            rd.start()
            rdmas.append(rd)
        for rd in rdmas:
            rd.wait_recv()

        own = acc_ref[pl.ds(my, 1), :, :]
        out_ref[:, :] = own[0] + jnp.sum(recv_ref[:, :, :], axis=0)
        for rd in rdmas:
            rd.wait_send()

    return pl.pallas_call(
        body,
        out_shape=jax.ShapeDtypeStruct((r, h), jnp.float32),
        in_specs=[pl.BlockSpec(memory_space=pltpu.VMEM)] * 4,
        out_specs=pl.BlockSpec(memory_space=pltpu.VMEM),
        scratch_shapes=[
            pltpu.VMEM((P, r, h), jnp.float32),
            pltpu.VMEM((P, r, h), jnp.float32),
            pltpu.SemaphoreType.DMA((P,)),
            pltpu.SemaphoreType.DMA((P,)),
        ],
        compiler_params=pltpu.CompilerParams(collective_id=0),
    )(x, router_W, route_idx, expert_W)


# device time: 12253 ns/iter; 1.0598x vs baseline; 1.0598x over previous
import jax
import jax.numpy as jnp
from jax import lax
from jax.experimental import pallas as pl
from jax.experimental.pallas import tpu as pltpu

P = 16
E_LOCAL = 2
N_EXPERTS = 32


def kernel(x, router_W, route_idx, expert_W):
    n, d = x.shape
    h = expert_W.shape[-1]
    r = n // P

    def body(x_ref, rw_ref, idx_ref, ew_ref, out_ref,
             acc_ref, recv_ref, send_sems, recv_sems):
        my = lax.axis_index("i")

        bsem = pltpu.get_barrier_semaphore()
        for j in range(1, P):
            t = lax.rem(my + j, P)
            pl.semaphore_signal(bsem, inc=1, device_id=(t,),
                                device_id_type=pl.DeviceIdType.MESH)

        xv = x_ref[:, :]
        scores = jnp.dot(xv, rw_ref[:, :], preferred_element_type=jnp.float32)
        m = jnp.max(scores, axis=-1, keepdims=True)
        p = jnp.exp(scores - m)
        p = p / jnp.sum(p, axis=-1, keepdims=True)
        idx0 = idx_ref[:, 0:1]
        idx1 = idx_ref[:, 1:2]
        e_iota = lax.broadcasted_iota(jnp.int32, (n, N_EXPERTS), 1)
        g0 = jnp.sum(jnp.where(idx0 == e_iota, p, 0.0), axis=1, keepdims=True)
        g1 = jnp.sum(jnp.where(idx1 == e_iota, p, 0.0), axis=1, keepdims=True)
        gsum = g0 + g1
        g0n = g0 / gsum
        g1n = g1 / gsum

        partial = jnp.zeros((n, h), jnp.float32)
        for le in range(E_LOCAL):
            ge = my * E_LOCAL + le
            w = jnp.where(idx0 == ge, g0n, 0.0) + jnp.where(idx1 == ge, g1n, 0.0)
            y = jnp.dot(xv, ew_ref[le, :, :], preferred_element_type=jnp.float32)
            partial = partial + w * y

        acc_ref[:, :, :] = partial.reshape(P, r, h)
        recv_ref[0, :, :] = jnp.zeros((r, h), jnp.float32)

        pl.semaphore_wait(bsem, P - 1)

        rdmas = []
        for j in range(1, P):
            t = lax.rem(my + j, P)
            rd = pltpu.make_async_remote_copy(
                src_ref=acc_ref.at[pl.ds(t, 1)],
                dst_ref=recv_ref.at[pl.ds(j, 1)],
                send_sem=send_sems.at[j],
                recv_sem=recv_sems.at[j],
                device_id=(t,),
                device_id_type=pl.DeviceIdType.MESH,
            )
            rd.start()
            rdmas.append(rd)
        for rd in rdmas:
            rd.wait_recv()

        own = acc_ref[pl.ds(my, 1), :, :]
        out_ref[:, :] = own[0] + jnp.sum(recv_ref[:, :, :], axis=0)
        for rd in rdmas:
            rd.wait_send()

    return pl.pallas_call(
        body,
        out_shape=jax.ShapeDtypeStruct((r, h), jnp.float32),
        in_specs=[pl.BlockSpec(memory_space=pltpu.VMEM)] * 4,
        out_specs=pl.BlockSpec(memory_space=pltpu.VMEM),
        scratch_shapes=[
            pltpu.VMEM((P, r, h), jnp.float32),
            pltpu.VMEM((P, r, h), jnp.float32),
            pltpu.SemaphoreType.DMA((P,)),
            pltpu.SemaphoreType.DMA((P,)),
        ],
        compiler_params=pltpu.CompilerParams(collective_id=0),
    )(x, router_W, route_idx, expert_W)
